# Initial kernel scaffold; baseline (speedup 1.0000x reference)
import jax
import jax.numpy as jnp
from jax.experimental import pallas as pl


def kernel(x, edge_index, Wg0, bg0, W0, b0, Wr0, Wg1, bg1, W1, b1, Wr1, Wl2, bl2, Wr2):
    raise NotImplementedError("write your pallas kernel here")


# speedup vs baseline: 2.9954x; 2.9954x over previous
import functools

import jax
import jax.numpy as jnp
from jax import lax
from jax.experimental import pallas as pl
from jax.experimental.pallas import tpu as pltpu
from jax.experimental.pallas import tpu_sc as plsc

_N = 10000
_E = 160000
_D = 256
_DOUT = 128
_NE = 8

_NC = 2
_NS = 16
_HALF = 128

_ROWS = 10240
_SLAB = _ROWS // _NS
_EPAD = 163840
_CHUNK = 128
_NCHUNK = _EPAD // _NS // _CHUNK
_CNTC = 128

_BN = 512
_NBLK = _ROWS // _BN


_MESH = plsc.VectorSubcoreMesh(core_axis_name="c", subcore_axis_name="s")


def _seg_sum_build():
  sums_t = jax.ShapeDtypeStruct((_NC, _ROWS, _HALF), jnp.float32)
  scratch = [
      pltpu.VMEM((_NCHUNK, _CHUNK), jnp.int32),
      pltpu.VMEM((_NCHUNK, _CHUNK), jnp.int32),
      pltpu.VMEM((_CHUNK, _HALF), jnp.float32),
      pltpu.VMEM_SHARED((_ROWS, _HALF), jnp.float32),
      pltpu.SemaphoreType.DMA,
  ]

  @functools.partial(pl.kernel, out_type=sums_t, mesh=_MESH,
                     scratch_types=scratch)
  def k(xs_hbm, src_hbm, dst_hbm, z_hbm, out_hbm, srcv, dstv, rowsv, acc,
        gsem):
    c = lax.axis_index("c")
    s = lax.axis_index("s")

    pltpu.sync_copy(z_hbm, acc.at[pl.ds(s * _SLAB, _SLAB)])
    pltpu.sync_copy(src_hbm.at[s], srcv)
    pltpu.sync_copy(dst_hbm.at[s], dstv)

    plsc.subcore_barrier()

    xc = xs_hbm.at[c]

    @pl.loop(0, _NCHUNK)
    def _(j):
      pltpu.async_copy(xc.at[srcv.at[j]], rowsv, gsem).wait()
      pltpu.sync_copy(rowsv, acc.at[dstv.at[j]], add=True)

    plsc.subcore_barrier()

    sl = pl.ds(s * _SLAB, _SLAB)
    pltpu.sync_copy(acc.at[sl], out_hbm.at[c].at[sl])

  return k


def _counts_build():
  cnt_t = jax.ShapeDtypeStruct((_ROWS, _CNTC), jnp.float32)
  scratch = [
      pltpu.VMEM((_NCHUNK, _CHUNK), jnp.int32),
      pltpu.VMEM((_CHUNK, _CNTC), jnp.float32),
      pltpu.VMEM_SHARED((_ROWS, _CNTC), jnp.float32),
  ]

  @functools.partial(pl.kernel, out_type=cnt_t, mesh=_MESH,
                     scratch_types=scratch)
  def k(dst_hbm, zc_hbm, ones_hbm, cnt_hbm, dstv, onesv, cntacc):
    c = lax.axis_index("c")
    s = lax.axis_index("s")

    @pl.when(c == 0)
    def _():
      pltpu.sync_copy(zc_hbm, cntacc.at[pl.ds(s * _SLAB, _SLAB)])
      pltpu.sync_copy(dst_hbm.at[s], dstv)
      pltpu.sync_copy(ones_hbm, onesv)

    plsc.subcore_barrier()

    @pl.when(c == 0)
    def _():
      @pl.loop(0, _NCHUNK)
      def _(j):
        pltpu.sync_copy(onesv, cntacc.at[dstv.at[j]], add=True)

    plsc.subcore_barrier()

    @pl.when(c == 0)
    def _():
      sl = pl.ds(s * _SLAB, _SLAB)
      pltpu.sync_copy(cntacc.at[sl], cnt_hbm.at[sl])

  return k


_seg_sum = _seg_sum_build()
_counts = _counts_build()


def _moe_body(sums_ref, cnt_ref, xprev_ref, Wg_ref, bg_ref, W_ref, b_ref,
              Wr_ref, out_ref, gstd_ref, *, relu):
  i = pl.program_id(0)
  cnt = cnt_ref[:, 0:1]
  recip = 1.0 / jnp.maximum(cnt, 1.0)
  h = jnp.concatenate([sums_ref[0], sums_ref[1]], axis=1) * recip
  logits = jnp.dot(h, Wg_ref[...], preferred_element_type=jnp.float32)
  logits = logits + bg_ref[...]
  col = lax.broadcasted_iota(jnp.int32, (_BN, _HALF), 1)
  lane_ok = col < _NE
  lm = jnp.where(lane_ok, logits, jnp.float32(-1e30))
  m = jnp.max(lm, axis=1, keepdims=True)
  p = jnp.exp(lm - m)
  p = p / jnp.sum(p, axis=1, keepdims=True)
  diff = jnp.where(lane_ok, p - jnp.float32(1.0 / _NE), 0.0)
  ssq = jnp.sum(diff * diff, axis=1, keepdims=True)
  std = jnp.sqrt(ssq * jnp.float32(1.0 / (_NE - 1)))
  rowid = lax.broadcasted_iota(jnp.int32, (_BN, 1), 0) + i * _BN
  gpart = jnp.sum(jnp.where(rowid < _N, std, 0.0))
  gstd_ref[...] = jnp.full((1, 1, _HALF), gpart, jnp.float32)
  eidx = jnp.min(jnp.where((lm >= m) & lane_ok, col, _NE), axis=1,
                 keepdims=True)
  xprev = jnp.concatenate([xprev_ref[0], xprev_ref[1]], axis=1)
  acc = jnp.dot(xprev, Wr_ref[...], preferred_element_type=jnp.float32)
  for e in range(_NE):
    sel = (eidx == e).astype(jnp.float32)
    pe = jnp.dot(h, W_ref[e], preferred_element_type=jnp.float32) + b_ref[e]
    acc = acc + sel * pe
  if relu:
    acc = jnp.maximum(acc, 0.0)
  out_ref[0] = acc[:, :_HALF]
  out_ref[1] = acc[:, _HALF:]


def _moe_tc(sums, cnt, xprev, Wgp, bgp, W, b, Wr, relu):
  return pl.pallas_call(
      functools.partial(_moe_body, relu=relu),
      grid=(_NBLK,),
      in_specs=[
          pl.BlockSpec((2, _BN, _HALF), lambda i: (0, i, 0)),
          pl.BlockSpec((_BN, _CNTC), lambda i: (i, 0)),
          pl.BlockSpec((2, _BN, _HALF), lambda i: (0, i, 0)),
          pl.BlockSpec((_D, _HALF), lambda i: (0, 0)),
          pl.BlockSpec((1, _HALF), lambda i: (0, 0)),
          pl.BlockSpec((_NE, _D, _D), lambda i: (0, 0, 0)),
          pl.BlockSpec((_NE, 1, _D), lambda i: (0, 0, 0)),
          pl.BlockSpec((_D, _D), lambda i: (0, 0)),
      ],
      out_specs=[
          pl.BlockSpec((2, _BN, _HALF), lambda i: (0, i, 0)),
          pl.BlockSpec((1, 1, _HALF), lambda i: (i, 0, 0)),
      ],
      out_shape=[
          jax.ShapeDtypeStruct((2, _ROWS, _HALF), jnp.float32),
          jax.ShapeDtypeStruct((_NBLK, 1, _HALF), jnp.float32),
      ],
  )(sums, cnt, xprev, Wgp, bgp, W, b, Wr)


def _fin_body(sums_ref, cnt_ref, x2_ref, Wl_ref, bl_ref, Wr_ref, out_ref):
  cnt = cnt_ref[:, 0:1]
  recip = 1.0 / jnp.maximum(cnt, 1.0)
  h = jnp.concatenate([sums_ref[0], sums_ref[1]], axis=1) * recip
  x2 = jnp.concatenate([x2_ref[0], x2_ref[1]], axis=1)
  out = jnp.dot(h, Wl_ref[...], preferred_element_type=jnp.float32)
  out = out + bl_ref[...]
  out = out + jnp.dot(x2, Wr_ref[...], preferred_element_type=jnp.float32)
  out_ref[...] = out


def _fin_tc(sums, cnt, x2, Wl, bl, Wr):
  return pl.pallas_call(
      _fin_body,
      grid=(_NBLK,),
      in_specs=[
          pl.BlockSpec((2, _BN, _HALF), lambda i: (0, i, 0)),
          pl.BlockSpec((_BN, _CNTC), lambda i: (i, 0)),
          pl.BlockSpec((2, _BN, _HALF), lambda i: (0, i, 0)),
          pl.BlockSpec((_D, _DOUT), lambda i: (0, 0)),
          pl.BlockSpec((1, _DOUT), lambda i: (0, 0)),
          pl.BlockSpec((_D, _DOUT), lambda i: (0, 0)),
      ],
      out_specs=pl.BlockSpec((_BN, _DOUT), lambda i: (i, 0)),
      out_shape=jax.ShapeDtypeStruct((_ROWS, _DOUT), jnp.float32),
  )(sums, cnt, x2, Wl, bl, Wr)


def kernel(x, edge_index, Wg0, bg0, W0, b0, Wr0, Wg1, bg1, W1, b1, Wr1,
           Wl2, bl2, Wr2):
  f32 = jnp.float32
  xp = jnp.zeros((_ROWS, _D), f32).at[:_N].set(x)
  xs = jnp.stack([xp[:, :_HALF], xp[:, _HALF:]])

  src = edge_index[0]
  dst = edge_index[1]
  pad = _EPAD - _E
  src2 = jnp.concatenate([src, jnp.zeros((pad,), jnp.int32)])
  src2 = src2.reshape(_NS, _NCHUNK, _CHUNK)
  dst2 = jnp.concatenate([dst, jnp.full((pad,), _N, jnp.int32)])
  dst2 = dst2.reshape(_NS, _NCHUNK, _CHUNK)

  z_rows = jnp.zeros((_SLAB, _HALF), f32)
  zc_rows = jnp.zeros((_SLAB, _CNTC), f32)
  ones_rows = jnp.ones((_CHUNK, _CNTC), f32)

  Wg0p = jnp.zeros((_D, _HALF), f32).at[:, :_NE].set(Wg0)
  bg0p = jnp.zeros((1, _HALF), f32).at[0, :_NE].set(bg0)
  Wg1p = jnp.zeros((_D, _HALF), f32).at[:, :_NE].set(Wg1)
  bg1p = jnp.zeros((1, _HALF), f32).at[0, :_NE].set(bg1)

  cnt = _counts(dst2, zc_rows, ones_rows)
  sums0 = _seg_sum(xs, src2, dst2, z_rows)
  x1s, g0 = _moe_tc(sums0, cnt, xs, Wg0p, bg0p, W0,
                    b0.reshape(_NE, 1, _D), Wr0, relu=True)
  sums1 = _seg_sum(x1s, src2, dst2, z_rows)
  x2s, g1 = _moe_tc(sums1, cnt, x1s, Wg1p, bg1p, W1,
                    b1.reshape(_NE, 1, _D), Wr1, relu=True)
  sums2 = _seg_sum(x2s, src2, dst2, z_rows)
  outp = _fin_tc(sums2, cnt, x2s, Wl2, bl2.reshape(1, _DOUT), Wr2)

  out = outp[:_N]
  gstd = (jnp.sum(g0[:, 0, 0]) + jnp.sum(g1[:, 0, 0])) / jnp.float32(2 * _N)
  return (out, gstd)
